# Initial kernel scaffold; baseline (speedup 1.0000x reference)
import jax
import jax.numpy as jnp
from jax.experimental import pallas as pl


def kernel(u_features, v_features, u_features_side, v_features_side, support_u_idx, support_v_idx, support_vals, support_ids, u_indices, v_indices, W_gcn, W_feat_u, b_feat_u, W_feat_v, b_feat_v, W_dense_u, W_dense_v, W_bilin, W_coef):
    raise NotImplementedError("write your pallas kernel here")


# speedup vs baseline: 2.2224x; 2.2224x over previous
import functools

import jax
import jax.numpy as jnp
from jax import lax
from jax.experimental import pallas as pl
from jax.experimental.pallas import tpu as pltpu
from jax.experimental.pallas import tpu_sc as plsc

NU = 10000
NV = 10000
NE = 160000
NL = 100000
DIN = 256
NSIDE = 32
FH = 64
H0 = 250
H1 = 75
NCLS = 5
NSUP = 5
NBAS = 2

CHUNK = H0 // NSUP
CW = 128
H1P = 128
NSPLIT = 5120
NHALF = 5248
DUMLOC = 5200
ROWS_PER_TILE = NHALF // 16
ZROWS = 2 * NHALF
NROWS_T = 10112
ECH = 128
EPAD = NE + ECH
PER_TILE_L = 3136
LPAD = 32 * PER_TILE_L
LCH = 112


def _gcn_mm_body(x_ref, w_ref, o_ref):
    o_ref[...] = jnp.dot(x_ref[...], w_ref[...],
                         preferred_element_type=jnp.float32)


def _gcn_matmul(feats, w_pad):
    bm = 400
    return pl.pallas_call(
        _gcn_mm_body,
        grid=(feats.shape[0] // bm,),
        in_specs=[
            pl.BlockSpec((bm, DIN), lambda i: (i, 0)),
            pl.BlockSpec((DIN, NSUP * CW), lambda i: (0, 0)),
        ],
        out_specs=pl.BlockSpec((bm, NSUP * CW), lambda i: (i, 0)),
        out_shape=jax.ShapeDtypeStruct((feats.shape[0], NSUP * CW),
                                       jnp.float32),
    )(feats, w_pad)


def _edge_kernel(tmp_flat, u_idx, v_idx, vals, bounds):
    mesh = plsc.VectorSubcoreMesh(core_axis_name="c", subcore_axis_name="s")

    @functools.partial(
        pl.kernel, mesh=mesh,
        out_type=[
            jax.ShapeDtypeStruct((NSUP, ZROWS, CW), jnp.float32),
            jax.ShapeDtypeStruct((NSUP, ZROWS, CW), jnp.float32),
        ],
        scratch_types=[
            pltpu.VMEM_SHARED((NHALF, CW), jnp.float32),
            pltpu.VMEM((ROWS_PER_TILE, CW), jnp.float32),
            pltpu.VMEM((ECH,), jnp.int32),
            pltpu.VMEM((ECH,), jnp.int32),
            pltpu.VMEM((ECH,), jnp.int32),
            pltpu.VMEM((ECH,), jnp.int32),
            pltpu.VMEM((ECH,), jnp.float32),
            pltpu.VMEM((ECH, CW), jnp.float32),
            pltpu.VMEM((16,), jnp.int32),
            pltpu.SemaphoreType.DMA,
        ],
    )
    def k(tmp_hbm, uidx_hbm, vidx_hbm, vals_hbm, bounds_hbm,
          zu_hbm, zv_hbm,
          acc, zbuf, sidx, didx, deff, gidx, vbuf, rows, bnd, sem):
        core = lax.axis_index("c")
        tid = lax.axis_index("s")
        half_lo = core * NSPLIT
        pltpu.sync_copy(bounds_hbm, bnd)

        def zrow(i, _):
            for q in range(CW // 16):
                zbuf[i, pl.ds(q * 16, 16)] = jnp.zeros((16,), jnp.float32)
            return 0
        lax.fori_loop(0, ROWS_PER_TILE, zrow, 0)

        iota = lax.iota(jnp.int32, 16)

        def run_side(src_hbm, dst_hbm, out_hbm, tab_off):
            b16 = bnd[pl.ds(0, 16)]
            for phase in range(NSUP):
                pltpu.sync_copy(zbuf, acc.at[pl.ds(tid * ROWS_PER_TILE,
                                                   ROWS_PER_TILE)])
                plsc.subcore_barrier()

                lo = b16[phase]
                hi = b16[phase + 1]
                cnt = hi - lo
                per = (cnt + 15) // 16
                my_lo = lo + tid * per
                my_hi = jnp.minimum(my_lo + per, hi)
                a0 = (my_lo // 8) * 8
                nch = (my_hi - a0 + (ECH - 1)) // ECH

                def chunk_body(j, _):
                    st = a0 + j * ECH
                    pltpu.sync_copy(src_hbm.at[pl.ds(st, ECH)], sidx)
                    pltpu.sync_copy(dst_hbm.at[pl.ds(st, ECH)], didx)
                    pltpu.sync_copy(vals_hbm.at[pl.ds(st, ECH)], vbuf)

                    def prep(g, _):
                        b = g * 16
                        sv = sidx[pl.ds(b, 16)]
                        gidx[pl.ds(b, 16)] = (sv + tab_off) * NSUP + phase
                        e = st + b + iota
                        lv = didx[pl.ds(b, 16)] - half_lo
                        ok = ((e >= my_lo) & (e < my_hi)
                              & (lv >= 0) & (lv < NSPLIT))
                        deff[pl.ds(b, 16)] = jnp.where(
                            ok, lv, jnp.full((16,), DUMLOC, jnp.int32))
                        return 0
                    lax.fori_loop(0, ECH // 16, prep, 0)

                    pltpu.async_copy(tmp_hbm.at[gidx], rows, sem).wait()

                    def scale(g, _):
                        b = g * 16
                        v16 = vbuf[pl.ds(b, 16)]
                        for i in range(16):
                            vv = lax.broadcast(v16[i], (16,))
                            for q in range(4):
                                rows[b + i, pl.ds(q * 16, 16)] = (
                                    rows[b + i, pl.ds(q * 16, 16)] * vv)
                        return 0
                    lax.fori_loop(0, ECH // 16, scale, 0)

                    pltpu.sync_copy(rows, acc.at[deff], add=True)
                    return 0
                lax.fori_loop(0, nch, chunk_body, 0)

                plsc.subcore_barrier()
                pltpu.sync_copy(
                    acc.at[pl.ds(tid * ROWS_PER_TILE, ROWS_PER_TILE)],
                    out_hbm.at[phase,
                               pl.ds(core * NHALF + tid * ROWS_PER_TILE,
                                     ROWS_PER_TILE)])
                plsc.subcore_barrier()

        run_side(vidx_hbm, uidx_hbm, zu_hbm, NU)
        run_side(uidx_hbm, vidx_hbm, zv_hbm, 0)

    return k(tmp_flat, u_idx, v_idx, vals, bounds)


def _dense_body(with_bilin, z_ref, side_ref, wdg_ref, wf_ref, b_ref,
                wds_ref, wb_ref, o_ref):
    bm = side_ref.shape[0]
    h = jnp.zeros((bm, H1P), jnp.float32)
    for s in range(NSUP):
        g = jnp.maximum(z_ref[s], 0.0)
        h = h + jnp.dot(g, wdg_ref[s], preferred_element_type=jnp.float32)
    f = jnp.maximum(
        jnp.dot(side_ref[...], wf_ref[...],
                preferred_element_type=jnp.float32) + b_ref[...], 0.0)
    h = h + jnp.dot(f, wds_ref[...], preferred_element_type=jnp.float32)
    if with_bilin:
        o_ref[:, 0:H1P] = jnp.dot(h, wb_ref[0],
                                  preferred_element_type=jnp.float32)
        o_ref[:, H1P:2 * H1P] = jnp.dot(h, wb_ref[1],
                                        preferred_element_type=jnp.float32)
    else:
        o_ref[...] = h


def _dense(z_c, side, wdg, wf, b2, wds, wb, with_bilin):
    bm = 128
    n = side.shape[0]
    ow = 2 * H1P if with_bilin else H1P
    zblk = lambda i: (0, jnp.where(i >= NSPLIT // bm, i + 1, i), 0)
    return pl.pallas_call(
        functools.partial(_dense_body, with_bilin),
        grid=(n // bm,),
        in_specs=[
            pl.BlockSpec((NSUP, bm, CW), zblk),
            pl.BlockSpec((bm, NSIDE), lambda i: (i, 0)),
            pl.BlockSpec((NSUP, CW, H1P), lambda i: (0, 0, 0)),
            pl.BlockSpec((NSIDE, FH), lambda i: (0, 0)),
            pl.BlockSpec((1, FH), lambda i: (0, 0)),
            pl.BlockSpec((FH, H1P), lambda i: (0, 0)),
            pl.BlockSpec((NBAS, H1P, H1P), lambda i: (0, 0, 0)),
        ],
        out_specs=pl.BlockSpec((bm, ow), lambda i: (i, 0)),
        out_shape=jax.ShapeDtypeStruct((n, ow), jnp.float32),
    )(z_c, side, wdg, wf, b2, wds, wb)


def _decode_kernel(hbu, hv, u_idx, v_idx):
    mesh = plsc.VectorSubcoreMesh(core_axis_name="c", subcore_axis_name="s")

    @functools.partial(
        pl.kernel, mesh=mesh,
        out_type=jax.ShapeDtypeStruct((LPAD, 16), jnp.float32),
        scratch_types=[
            pltpu.VMEM((LCH,), jnp.int32),
            pltpu.VMEM((LCH,), jnp.int32),
            pltpu.VMEM((LCH, 2 * H1P), jnp.float32),
            pltpu.VMEM((LCH, H1P), jnp.float32),
            pltpu.VMEM((LCH, 16), jnp.float32),
            pltpu.SemaphoreType.DMA,
        ],
    )
    def k(hbu_hbm, hv_hbm, uidx_hbm, vidx_hbm, out_hbm,
          ui, vi, ru, rv, ob, sem):
        core = lax.axis_index("c")
        tid = lax.axis_index("s")
        wid = tid * 2 + core
        base = wid * PER_TILE_L
        iota = lax.iota(jnp.int32, 16)

        def chunk_body(j, _):
            st = base + j * LCH
            pltpu.sync_copy(uidx_hbm.at[pl.ds(st, LCH)], ui)
            pltpu.sync_copy(vidx_hbm.at[pl.ds(st, LCH)], vi)
            cp1 = pltpu.async_copy(hbu_hbm.at[ui], ru, sem)
            cp2 = pltpu.async_copy(hv_hbm.at[vi], rv, sem)
            cp1.wait()
            cp2.wait()

            def hsum(v):
                for m in (8, 4, 2, 1):
                    v = v + v[jnp.bitwise_xor(iota, m)]
                return v

            def link(l, _):
                acc0 = jnp.zeros((16,), jnp.float32)
                acc1 = jnp.zeros((16,), jnp.float32)
                for q in range(H1P // 16):
                    v = rv[l, pl.ds(q * 16, 16)]
                    acc0 = acc0 + ru[l, pl.ds(q * 16, 16)] * v
                    acc1 = acc1 + ru[l, pl.ds(H1P + q * 16, 16)] * v
                row = jnp.where(iota == 0, hsum(acc0),
                                jnp.where(iota == 1, hsum(acc1),
                                          jnp.zeros((16,), jnp.float32)))
                ob[l, pl.ds(0, 16)] = row
                return 0
            lax.fori_loop(0, LCH, link, 0)

            pltpu.sync_copy(ob, out_hbm.at[pl.ds(st, LCH)])
            return 0
        lax.fori_loop(0, PER_TILE_L // LCH, chunk_body, 0)

    return k(hbu, hv, u_idx, v_idx)


def _coef_body(b_ref, w_ref, o_ref):
    o_ref[...] = jnp.dot(b_ref[...], w_ref[...],
                         preferred_element_type=jnp.float32)


def _coef(basis, wc_pad):
    bm = 400
    return pl.pallas_call(
        _coef_body,
        grid=(NL // bm,),
        in_specs=[
            pl.BlockSpec((bm, 16), lambda i: (i, 0)),
            pl.BlockSpec((16, NCLS), lambda i: (0, 0)),
        ],
        out_specs=pl.BlockSpec((bm, NCLS), lambda i: (i, 0)),
        out_shape=jax.ShapeDtypeStruct((NL, NCLS), jnp.float32),
    )(basis, wc_pad)


def kernel(u_features, v_features, u_features_side, v_features_side,
           support_u_idx, support_v_idx, support_vals, support_ids,
           u_indices, v_indices,
           W_gcn, W_feat_u, b_feat_u, W_feat_v, b_feat_v,
           W_dense_u, W_dense_v, W_bilin, W_coef):
    f32 = jnp.float32
    i32 = jnp.int32

    feats = jnp.concatenate([u_features, v_features], axis=0)
    w_pad = jnp.pad(W_gcn.reshape(DIN, NSUP, CHUNK),
                    ((0, 0), (0, 0), (0, CW - CHUNK))).reshape(DIN, NSUP * CW)

    sid = support_ids.astype(i32)
    bounds = jnp.searchsorted(sid, jnp.arange(NSUP + 1, dtype=i32)).astype(i32)
    bounds = jnp.pad(bounds, (0, 16 - (NSUP + 1)))
    uix = jnp.pad(support_u_idx.astype(i32), (0, EPAD - NE))
    vix = jnp.pad(support_v_idx.astype(i32), (0, EPAD - NE))
    vls = jnp.pad(support_vals.astype(f32), (0, EPAD - NE))

    def split_wd(wd):
        g = jnp.pad(wd[:H0].reshape(NSUP, CHUNK, H1),
                    ((0, 0), (0, CW - CHUNK), (0, H1P - H1)))
        s = jnp.pad(wd[H0:], ((0, 0), (0, H1P - H1)))
        return g, s

    wdg_u, wds_u = split_wd(W_dense_u)
    wdg_v, wds_v = split_wd(W_dense_v)
    wb_pad = jnp.pad(W_bilin, ((0, 0), (0, H1P - H1), (0, H1P - H1)))
    bu2 = b_feat_u.reshape(1, FH)
    bv2 = b_feat_v.reshape(1, FH)
    wc_pad = jnp.pad(W_coef, ((0, 16 - NBAS), (0, 0)))

    lu = jnp.pad(u_indices.astype(i32), (0, LPAD - NL))
    lv = jnp.pad(v_indices.astype(i32), (0, LPAD - NL))

    tmp_flat = _gcn_matmul(feats, w_pad).reshape(2 * NU * NSUP, CW)

    zu_c, zv_c = _edge_kernel(tmp_flat, uix, vix, vls, bounds)

    side_u = jnp.pad(u_features_side, ((0, NROWS_T - NU), (0, 0)))
    side_v = jnp.pad(v_features_side, ((0, NROWS_T - NV), (0, 0)))
    hbu = _dense(zu_c, side_u, wdg_u, W_feat_u, bu2, wds_u, wb_pad, True)
    hv = _dense(zv_c, side_v, wdg_v, W_feat_v, bv2, wds_v, wb_pad, False)

    basis = _decode_kernel(hbu, hv, lu, lv)

    return _coef(basis, wc_pad)


# speedup vs baseline: 2.9092x; 1.3090x over previous
import functools

import jax
import jax.numpy as jnp
from jax import lax
from jax.experimental import pallas as pl
from jax.experimental.pallas import tpu as pltpu
from jax.experimental.pallas import tpu_sc as plsc

NU = 10000
NV = 10000
NE = 160000
NL = 100000
DIN = 256
NSIDE = 32
FH = 64
H0 = 250
H1 = 75
NCLS = 5
NSUP = 5
NBAS = 2

CHUNK = H0 // NSUP
CW = 128
H1P = 128
ACC_W = 64
NPAD = 10112
DUMMY = 10000
ROWS_PER_TILE = NPAD // 16
NROWS_T = 10112
ECH = 128
EPAD = NE + ECH
PER_TILE_L = 3136
LPAD = 32 * PER_TILE_L
LCH = 112


def _gcn_mm_body(x_ref, w_ref, o_ref):
    o_ref[...] = jnp.dot(x_ref[...], w_ref[...],
                         preferred_element_type=jnp.float32)


def _gcn_matmul(feats, w_pad):
    bm = 400
    return pl.pallas_call(
        _gcn_mm_body,
        grid=(feats.shape[0] // bm,),
        in_specs=[
            pl.BlockSpec((bm, DIN), lambda i: (i, 0)),
            pl.BlockSpec((DIN, NSUP * CW), lambda i: (0, 0)),
        ],
        out_specs=pl.BlockSpec((bm, NSUP * CW), lambda i: (i, 0)),
        out_shape=jax.ShapeDtypeStruct((feats.shape[0], NSUP * CW),
                                       jnp.float32),
    )(feats, w_pad)


def _edge_kernel(tmp_flat, u_idx, v_idx, vals, bounds, zeros_hbm):
    mesh = plsc.VectorSubcoreMesh(core_axis_name="c", subcore_axis_name="s")

    @functools.partial(
        pl.kernel, mesh=mesh,
        out_type=jax.ShapeDtypeStruct((2, NSUP, NPAD, ACC_W), jnp.float32),
        scratch_types=[
            pltpu.VMEM_SHARED((NPAD, ACC_W), jnp.float32),
            pltpu.VMEM((ECH,), jnp.int32),
            pltpu.VMEM((ECH,), jnp.int32),
            pltpu.VMEM((ECH,), jnp.int32),
            pltpu.VMEM((ECH,), jnp.int32),
            pltpu.VMEM((ECH,), jnp.float32),
            pltpu.VMEM((ECH, CW), jnp.float32),
            pltpu.VMEM((ECH, ACC_W), jnp.float32),
            pltpu.VMEM((16,), jnp.int32),
            pltpu.SemaphoreType.DMA,
        ],
    )
    def k(tmp_hbm, uidx_hbm, vidx_hbm, vals_hbm, bounds_hbm, zz_hbm, z_hbm,
          acc, ubuf, vibuf, deff, gidx, vbuf, rows, rows64, bnd, sem):
        core = lax.axis_index("c")
        tid = lax.axis_index("s")
        pltpu.sync_copy(bounds_hbm, bnd)

        iota = lax.iota(jnp.int32, 16)
        cv = lax.broadcast(core, (16,))
        tab_off = NU * (1 - core)

        b16 = bnd[pl.ds(0, 16)]
        for phase in range(NSUP):
            pltpu.sync_copy(zz_hbm, acc.at[pl.ds(tid * ROWS_PER_TILE,
                                                 ROWS_PER_TILE)])
            plsc.subcore_barrier()

            lo = b16[phase]
            hi = b16[phase + 1]
            cnt = hi - lo
            per = (cnt + 15) // 16
            my_lo = lo + tid * per
            my_hi = jnp.minimum(my_lo + per, hi)
            a0 = (my_lo // 8) * 8
            nch = (my_hi - a0 + (ECH - 1)) // ECH

            def chunk_body(j, _):
                st = a0 + j * ECH
                pltpu.sync_copy(uidx_hbm.at[pl.ds(st, ECH)], ubuf)
                pltpu.sync_copy(vidx_hbm.at[pl.ds(st, ECH)], vibuf)
                pltpu.sync_copy(vals_hbm.at[pl.ds(st, ECH)], vbuf)

                def prep(g, _):
                    b = g * 16
                    uu = ubuf[pl.ds(b, 16)]
                    vv = vibuf[pl.ds(b, 16)]
                    src = vv + (uu - vv) * cv
                    dst = uu + (vv - uu) * cv
                    gidx[pl.ds(b, 16)] = (src + tab_off) * NSUP + phase
                    e = st + b + iota
                    ok = (e >= my_lo) & (e < my_hi)
                    deff[pl.ds(b, 16)] = jnp.where(
                        ok, dst, jnp.full((16,), DUMMY, jnp.int32))
                    return 0
                lax.fori_loop(0, ECH // 16, prep, 0)

                pltpu.async_copy(tmp_hbm.at[gidx], rows, sem).wait()

                def scale(g, _):
                    b = g * 16
                    v16 = vbuf[pl.ds(b, 16)]
                    for i in range(16):
                        vv = lax.broadcast(v16[i], (16,))
                        for q in range(ACC_W // 16):
                            rows64[b + i, pl.ds(q * 16, 16)] = (
                                rows[b + i, pl.ds(q * 16, 16)] * vv)
                    return 0
                lax.fori_loop(0, ECH // 16, scale, 0)

                pltpu.sync_copy(rows64, acc.at[deff], add=True)
                return 0
            lax.fori_loop(0, nch, chunk_body, 0)

            plsc.subcore_barrier()
            pltpu.sync_copy(
                acc.at[pl.ds(tid * ROWS_PER_TILE, ROWS_PER_TILE)],
                z_hbm.at[core, phase, pl.ds(tid * ROWS_PER_TILE,
                                            ROWS_PER_TILE)])
            plsc.subcore_barrier()

    return k(tmp_flat, u_idx, v_idx, vals, bounds, zeros_hbm)


def _dense_body(with_bilin, z_ref, side_ref, wdg_ref, wf_ref, b_ref,
                wds_ref, wb_ref, o_ref):
    bm = side_ref.shape[0]
    h = jnp.zeros((bm, H1P), jnp.float32)
    for s in range(NSUP):
        g = jnp.maximum(z_ref[0, s], 0.0)
        h = h + jnp.dot(g, wdg_ref[s], preferred_element_type=jnp.float32)
    f = jnp.maximum(
        jnp.dot(side_ref[...], wf_ref[...],
                preferred_element_type=jnp.float32) + b_ref[...], 0.0)
    h = h + jnp.dot(f, wds_ref[...], preferred_element_type=jnp.float32)
    if with_bilin:
        o_ref[:, 0:H1P] = jnp.dot(h, wb_ref[0],
                                  preferred_element_type=jnp.float32)
        o_ref[:, H1P:2 * H1P] = jnp.dot(h, wb_ref[1],
                                        preferred_element_type=jnp.float32)
    else:
        o_ref[...] = h


def _dense(z_all, sd, side, wdg, wf, b2, wds, wb, with_bilin):
    bm = 128
    n = side.shape[0]
    ow = 2 * H1P if with_bilin else H1P
    return pl.pallas_call(
        functools.partial(_dense_body, with_bilin),
        grid=(n // bm,),
        in_specs=[
            pl.BlockSpec((1, NSUP, bm, ACC_W), lambda i: (sd, 0, i, 0)),
            pl.BlockSpec((bm, NSIDE), lambda i: (i, 0)),
            pl.BlockSpec((NSUP, ACC_W, H1P), lambda i: (0, 0, 0)),
            pl.BlockSpec((NSIDE, FH), lambda i: (0, 0)),
            pl.BlockSpec((1, FH), lambda i: (0, 0)),
            pl.BlockSpec((FH, H1P), lambda i: (0, 0)),
            pl.BlockSpec((NBAS, H1P, H1P), lambda i: (0, 0, 0)),
        ],
        out_specs=pl.BlockSpec((bm, ow), lambda i: (i, 0)),
        out_shape=jax.ShapeDtypeStruct((n, ow), jnp.float32),
    )(z_all, side, wdg, wf, b2, wds, wb)


def _decode_kernel(hbu, hv, u_idx, v_idx):
    mesh = plsc.VectorSubcoreMesh(core_axis_name="c", subcore_axis_name="s")

    @functools.partial(
        pl.kernel, mesh=mesh,
        out_type=jax.ShapeDtypeStruct((LPAD, 16), jnp.float32),
        scratch_types=[
            pltpu.VMEM((LCH,), jnp.int32),
            pltpu.VMEM((LCH,), jnp.int32),
            pltpu.VMEM((LCH, 2 * H1P), jnp.float32),
            pltpu.VMEM((LCH, H1P), jnp.float32),
            pltpu.VMEM((LCH, 16), jnp.float32),
            pltpu.SemaphoreType.DMA,
        ],
    )
    def k(hbu_hbm, hv_hbm, uidx_hbm, vidx_hbm, out_hbm,
          ui, vi, ru, rv, ob, sem):
        core = lax.axis_index("c")
        tid = lax.axis_index("s")
        wid = tid * 2 + core
        base = wid * PER_TILE_L
        iota = lax.iota(jnp.int32, 16)

        def chunk_body(j, _):
            st = base + j * LCH
            pltpu.sync_copy(uidx_hbm.at[pl.ds(st, LCH)], ui)
            pltpu.sync_copy(vidx_hbm.at[pl.ds(st, LCH)], vi)
            cp1 = pltpu.async_copy(hbu_hbm.at[ui], ru, sem)
            cp2 = pltpu.async_copy(hv_hbm.at[vi], rv, sem)
            cp1.wait()
            cp2.wait()

            def hsum(v):
                for m in (8, 4, 2, 1):
                    v = v + v[jnp.bitwise_xor(iota, m)]
                return v

            def link(l, _):
                acc0 = jnp.zeros((16,), jnp.float32)
                acc1 = jnp.zeros((16,), jnp.float32)
                for q in range(H1P // 16):
                    v = rv[l, pl.ds(q * 16, 16)]
                    acc0 = acc0 + ru[l, pl.ds(q * 16, 16)] * v
                    acc1 = acc1 + ru[l, pl.ds(H1P + q * 16, 16)] * v
                row = jnp.where(iota == 0, hsum(acc0),
                                jnp.where(iota == 1, hsum(acc1),
                                          jnp.zeros((16,), jnp.float32)))
                ob[l, pl.ds(0, 16)] = row
                return 0
            lax.fori_loop(0, LCH, link, 0)

            pltpu.sync_copy(ob, out_hbm.at[pl.ds(st, LCH)])
            return 0
        lax.fori_loop(0, PER_TILE_L // LCH, chunk_body, 0)

    return k(hbu, hv, u_idx, v_idx)


def _coef_body(b_ref, w_ref, o_ref):
    o_ref[...] = jnp.dot(b_ref[...], w_ref[...],
                         preferred_element_type=jnp.float32)


def _coef(basis, wc_pad):
    bm = 400
    return pl.pallas_call(
        _coef_body,
        grid=(NL // bm,),
        in_specs=[
            pl.BlockSpec((bm, 16), lambda i: (i, 0)),
            pl.BlockSpec((16, NCLS), lambda i: (0, 0)),
        ],
        out_specs=pl.BlockSpec((bm, NCLS), lambda i: (i, 0)),
        out_shape=jax.ShapeDtypeStruct((NL, NCLS), jnp.float32),
    )(basis, wc_pad)


def kernel(u_features, v_features, u_features_side, v_features_side,
           support_u_idx, support_v_idx, support_vals, support_ids,
           u_indices, v_indices,
           W_gcn, W_feat_u, b_feat_u, W_feat_v, b_feat_v,
           W_dense_u, W_dense_v, W_bilin, W_coef):
    f32 = jnp.float32
    i32 = jnp.int32

    feats = jnp.concatenate([u_features, v_features], axis=0)
    w_pad = jnp.pad(W_gcn.reshape(DIN, NSUP, CHUNK),
                    ((0, 0), (0, 0), (0, CW - CHUNK))).reshape(DIN, NSUP * CW)

    sid = support_ids.astype(i32)
    bounds = jnp.searchsorted(sid, jnp.arange(NSUP + 1, dtype=i32)).astype(i32)
    bounds = jnp.pad(bounds, (0, 16 - (NSUP + 1)))
    uix = jnp.pad(support_u_idx.astype(i32), (0, EPAD - NE))
    vix = jnp.pad(support_v_idx.astype(i32), (0, EPAD - NE))
    vls = jnp.pad(support_vals.astype(f32), (0, EPAD - NE))

    def split_wd(wd):
        g = jnp.pad(wd[:H0].reshape(NSUP, CHUNK, H1),
                    ((0, 0), (0, ACC_W - CHUNK), (0, H1P - H1)))
        s = jnp.pad(wd[H0:], ((0, 0), (0, H1P - H1)))
        return g, s

    wdg_u, wds_u = split_wd(W_dense_u)
    wdg_v, wds_v = split_wd(W_dense_v)
    wb_pad = jnp.pad(W_bilin, ((0, 0), (0, H1P - H1), (0, H1P - H1)))
    bu2 = b_feat_u.reshape(1, FH)
    bv2 = b_feat_v.reshape(1, FH)
    wc_pad = jnp.pad(W_coef, ((0, 16 - NBAS), (0, 0)))

    lu = jnp.pad(u_indices.astype(i32), (0, LPAD - NL))
    lv = jnp.pad(v_indices.astype(i32), (0, LPAD - NL))

    tmp_flat = _gcn_matmul(feats, w_pad).reshape(2 * NU * NSUP, CW)

    zzero = jnp.zeros((ROWS_PER_TILE, ACC_W), jnp.float32)
    z_all = _edge_kernel(tmp_flat, uix, vix, vls, bounds, zzero)

    side_u = jnp.pad(u_features_side, ((0, NROWS_T - NU), (0, 0)))
    side_v = jnp.pad(v_features_side, ((0, NROWS_T - NV), (0, 0)))
    hbu = _dense(z_all, 0, side_u, wdg_u, W_feat_u, bu2, wds_u, wb_pad, True)
    hv = _dense(z_all, 1, side_v, wdg_v, W_feat_v, bv2, wds_v, wb_pad, False)

    basis = _decode_kernel(hbu, hv, lu, lv)

    return _coef(basis, wc_pad)
